# baseline (speedup 1.0000x reference)
import functools
import math

import jax
import jax.numpy as jnp
from jax import lax
from jax.experimental import pallas as pl
from jax.experimental.pallas import tpu as pltpu
from jax.experimental.pallas import tpu_sc as plsc

D = 64
SCALE = math.sqrt(D)
V = 1000000
B = 16384
S = 50

NUM_CORES = 2
NUM_SUBCORES = 16
NW = NUM_CORES * NUM_SUBCORES
LANES = 16

W = 512
NSTRIPE = V // W
TAIL = V - NSTRIPE * W
K1_ITERS = (NSTRIPE + NW - 1) // NW

BSTRIPE = B // NW
GCHUNK = 128


def _transpose_stripe(in_v, o_v, stride, n_rows):
    def row_body(w, c):
        for cg in range(D // LANES):
            cvec = (lax.iota(jnp.int32, LANES) + cg * LANES) * stride + w
            o_v[pl.ds(w * D + cg * LANES, LANES)] = plsc.load_gather(
                in_v, [cvec])
        return c

    lax.fori_loop(0, n_rows, row_body, 0, unroll=4)


def _k1_body(embt_hbm, lin_hbm, in_v, o_v, sem):
    wid = lax.axis_index("s") * NUM_CORES + lax.axis_index("c")

    def do_stripe(i0, n):
        copies = [
            pltpu.async_copy(
                embt_hbm.at[c, pl.ds(i0, n)],
                in_v.at[pl.ds(c * n, n)],
                sem,
            )
            for c in range(D)
        ]
        for cp in copies:
            cp.wait()
        _transpose_stripe(in_v, o_v, n, n)
        pltpu.sync_copy(
            o_v.at[pl.ds(0, n * D)], lin_hbm.at[pl.ds(i0 * D, n * D)])

    def stripe_body(k, carry):
        t = wid + k * NW

        @pl.when(t < NSTRIPE)
        def _():
            do_stripe(t * W, W)

        return carry

    lax.fori_loop(0, K1_ITERS, stripe_body, 0)

    @pl.when(wid == 0)
    def _():
        do_stripe(NSTRIPE * W, TAIL)


def _k2_body(xt_hbm, lin_hbm, out_hbm, idx_v, g_v, o_v, sem, sem2):
    wid = lax.axis_index("s") * NUM_CORES + lax.axis_index("c")
    b0 = wid * BSTRIPE
    pltpu.sync_copy(xt_hbm.at[:, pl.ds(b0, BSTRIPE)], idx_v)

    def s_body(s, carry):
        copies = [
            pltpu.async_copy(
                lin_hbm.at[idx_v.at[s, pl.ds(h * GCHUNK, GCHUNK)]],
                g_v.at[pl.ds(h * GCHUNK, GCHUNK)],
                sem,
            )
            for h in range(BSTRIPE // GCHUNK)
        ]
        for cp in copies:
            cp.wait()

        def row_body(j, c2):
            for cg in range(D // LANES):
                vals = g_v[j, pl.ds(cg * LANES, LANES)] * SCALE
                ovec = (lax.iota(jnp.int32, LANES) + cg * LANES) * BSTRIPE + j
                plsc.store_scatter(o_v, [ovec], vals)
            return c2

        lax.fori_loop(0, BSTRIPE, row_body, 0, unroll=4)

        wb = [
            pltpu.async_copy(
                o_v.at[pl.ds(c * BSTRIPE, BSTRIPE)],
                out_hbm.at[s, c, pl.ds(b0, BSTRIPE)],
                sem2,
            )
            for c in range(D)
        ]
        for cp in wb:
            cp.wait()
        return carry

    lax.fori_loop(0, S, s_body, 0)


@jax.jit
def _lookup(xt, embt):
    mesh = plsc.VectorSubcoreMesh(core_axis_name="c", subcore_axis_name="s")
    k1 = pl.kernel(
        _k1_body,
        out_type=jax.ShapeDtypeStruct((V * D,), jnp.float32),
        mesh=mesh,
        scratch_types=[
            pltpu.VMEM((D * W,), jnp.float32),
            pltpu.VMEM((W * D,), jnp.float32),
            pltpu.SemaphoreType.DMA,
        ],
        compiler_params=pltpu.CompilerParams(
            use_tc_tiling_on_sc=False, needs_layout_passes=False),
    )
    lin = k1(embt).reshape(V, D)

    k2 = pl.kernel(
        _k2_body,
        out_type=jax.ShapeDtypeStruct((S, D, B), jnp.float32),
        mesh=mesh,
        scratch_types=[
            pltpu.VMEM((S, BSTRIPE), jnp.int32),
            pltpu.VMEM((BSTRIPE, D), jnp.float32),
            pltpu.VMEM((D * BSTRIPE,), jnp.float32),
            pltpu.SemaphoreType.DMA,
            pltpu.SemaphoreType.DMA,
        ],
        compiler_params=pltpu.CompilerParams(
            use_tc_tiling_on_sc=False, needs_layout_passes=False),
    )
    return k2(xt, lin)


def kernel(x, emb):
    xt = x.T.astype(jnp.int32)
    embt = emb.T
    out3 = _lookup(xt, embt)
    return out3.transpose(2, 0, 1)


# speedup vs baseline: 4.1272x; 4.1272x over previous
import functools
import math

import jax
import jax.numpy as jnp
from jax import lax
from jax.experimental import pallas as pl
from jax.experimental.pallas import tpu as pltpu
from jax.experimental.pallas import tpu_sc as plsc

D = 64
SCALE = math.sqrt(D)
V = 1000000
B = 16384
S = 50

NUM_CORES = 2
NUM_SUBCORES = 16
NW = NUM_CORES * NUM_SUBCORES
LANES = 16

BSTRIPE = B // NW
GCHUNK = 128


def _body(xt_hbm, emb_hbm, out_hbm, idx_v, g_v, o_v, sem, sem2, semw):
    wid = lax.axis_index("s") * NUM_CORES + lax.axis_index("c")
    b0 = wid * BSTRIPE
    pltpu.sync_copy(xt_hbm.at[:, pl.ds(b0, BSTRIPE)], idx_v)

    def gather_s(s, buf, sm):
        return [
            pltpu.async_copy(
                emb_hbm.at[idx_v.at[s, pl.ds(h * GCHUNK, GCHUNK)]],
                g_v.at[buf, pl.ds(h * GCHUNK, GCHUNK)],
                sm,
            )
            for h in range(BSTRIPE // GCHUNK)
        ]

    def transpose_writeback(buf, s):
        def row_body(j, c2):
            for cg in range(D // LANES):
                vals = g_v[buf, j, pl.ds(cg * LANES, LANES)] * SCALE
                cvec = lax.iota(jnp.int32, LANES) + cg * LANES
                jvec = jnp.full((LANES,), j, jnp.int32)
                plsc.store_scatter(o_v, [cvec, jvec], vals)
            return c2

        lax.fori_loop(0, BSTRIPE, row_body, 0, unroll=8)
        pltpu.sync_copy(o_v, out_hbm.at[s, :, pl.ds(b0, BSTRIPE)])

    for cp in gather_s(0, 0, sem):
        cp.wait()

    def s_body(sh, c3):
        s0 = sh * 2

        @pl.when(s0 + 1 < S)
        def _():
            cps = gather_s(s0 + 1, 1, sem2)
            transpose_writeback(0, s0)
            for cp in cps:
                cp.wait()

        @pl.when(s0 + 2 < S)
        def _():
            cps = gather_s(s0 + 2, 0, sem)
            transpose_writeback(1, s0 + 1)
            for cp in cps:
                cp.wait()

        @pl.when(s0 + 1 == S)
        def _():
            transpose_writeback(0, s0)

        @pl.when(s0 + 2 == S)
        def _():
            transpose_writeback(1, s0 + 1)

        return c3

    lax.fori_loop(0, (S + 1) // 2, s_body, 0)


@jax.jit
def _lookup(xt, emb):
    mesh = plsc.VectorSubcoreMesh(core_axis_name="c", subcore_axis_name="s")
    f = pl.kernel(
        _body,
        out_type=jax.ShapeDtypeStruct((S, D, B), jnp.float32),
        mesh=mesh,
        scratch_types=[
            pltpu.VMEM((S, BSTRIPE), jnp.int32),
            pltpu.VMEM((2, BSTRIPE, D), jnp.float32),
            pltpu.VMEM((D, BSTRIPE), jnp.float32),
            pltpu.SemaphoreType.DMA,
            pltpu.SemaphoreType.DMA,
            pltpu.SemaphoreType.DMA,
        ],
        compiler_params=pltpu.CompilerParams(
            use_tc_tiling_on_sc=False, needs_layout_passes=False),
    )
    return f(xt, emb)


def kernel(x, emb):
    xt = x.T.astype(jnp.int32)
    out3 = _lookup(xt, emb)
    return out3.transpose(2, 0, 1)


# speedup vs baseline: 6.0478x; 1.4654x over previous
import functools
import math

import jax
import jax.numpy as jnp
from jax import lax
from jax.experimental import pallas as pl
from jax.experimental.pallas import tpu as pltpu
from jax.experimental.pallas import tpu_sc as plsc

D = 64
SCALE = math.sqrt(D)
V = 1000000
B = 16384
S = 50

NUM_CORES = 2
NUM_SUBCORES = 16
NW = NUM_CORES * NUM_SUBCORES
LANES = 16

BSTRIPE = B // NW
GCHUNK = 128


def _body(xt_hbm, emb_hbm, out_hbm, idx_v, g_v, o_v, sem, sem2, semw):
    wid = lax.axis_index("s") * NUM_CORES + lax.axis_index("c")
    b0 = wid * BSTRIPE
    pltpu.sync_copy(xt_hbm.at[:, pl.ds(b0, BSTRIPE)], idx_v)

    def gather_s(s, buf, sm):
        return [
            pltpu.async_copy(
                emb_hbm.at[idx_v.at[s, pl.ds(h * GCHUNK, GCHUNK)]],
                g_v.at[buf, pl.ds(h * GCHUNK, GCHUNK)],
                sm,
            )
            for h in range(BSTRIPE // GCHUNK)
        ]

    def transpose_writeback(buf, s):
        def row_body(j, c2):
            for cg in range(D // LANES):
                vals = g_v[buf, j, pl.ds(cg * LANES, LANES)] * SCALE
                cvec = lax.iota(jnp.int32, LANES) + cg * LANES
                jvec = jnp.full((LANES,), j, jnp.int32)
                plsc.store_scatter(o_v, [cvec, jvec], vals)
            return c2

        lax.fori_loop(0, BSTRIPE, row_body, 0, unroll=8)
        pltpu.sync_copy(
            o_v.at[:, pl.ds(0, BSTRIPE)],
            out_hbm.at[s, :, pl.ds(b0, BSTRIPE)])

    for cp in gather_s(0, 0, sem):
        cp.wait()

    def s_body(sh, c3):
        s0 = sh * 2

        @pl.when(s0 + 1 < S)
        def _():
            cps = gather_s(s0 + 1, 1, sem2)
            transpose_writeback(0, s0)
            for cp in cps:
                cp.wait()

        @pl.when(s0 + 2 < S)
        def _():
            cps = gather_s(s0 + 2, 0, sem)
            transpose_writeback(1, s0 + 1)
            for cp in cps:
                cp.wait()

        @pl.when(s0 + 1 == S)
        def _():
            transpose_writeback(0, s0)

        @pl.when(s0 + 2 == S)
        def _():
            transpose_writeback(1, s0 + 1)

        return c3

    lax.fori_loop(0, (S + 1) // 2, s_body, 0)


@jax.jit
def _lookup(xt, emb):
    mesh = plsc.VectorSubcoreMesh(core_axis_name="c", subcore_axis_name="s")
    f = pl.kernel(
        _body,
        out_type=jax.ShapeDtypeStruct((S, D, B), jnp.float32),
        mesh=mesh,
        scratch_types=[
            pltpu.VMEM((S, BSTRIPE), jnp.int32),
            pltpu.VMEM((2, BSTRIPE, D), jnp.float32),
            pltpu.VMEM((D, BSTRIPE + 1), jnp.float32),
            pltpu.SemaphoreType.DMA,
            pltpu.SemaphoreType.DMA,
            pltpu.SemaphoreType.DMA,
        ],
        compiler_params=pltpu.CompilerParams(
            use_tc_tiling_on_sc=False, needs_layout_passes=False),
    )
    return f(xt, emb)


def kernel(x, emb):
    xt = x.T.astype(jnp.int32)
    out3 = _lookup(xt, emb)
    return out3.transpose(2, 0, 1)


# speedup vs baseline: 6.2479x; 1.0331x over previous
import functools
import math

import jax
import jax.numpy as jnp
from jax import lax
from jax.experimental import pallas as pl
from jax.experimental.pallas import tpu as pltpu
from jax.experimental.pallas import tpu_sc as plsc

D = 64
SCALE = math.sqrt(D)
V = 1000000
B = 16384
S = 50

NUM_CORES = 2
NUM_SUBCORES = 16
NW = NUM_CORES * NUM_SUBCORES
LANES = 16

BSTRIPE = B // NW
GCHUNK = 128


HSTRIPE = BSTRIPE // 2


def _body(xt_hbm, emb_hbm, out_hbm, idx_v, g_v, o_v,
          sem, sem2, semw, semw2):
    wid = lax.axis_index("s") * NUM_CORES + lax.axis_index("c")
    b0 = wid * BSTRIPE
    pltpu.sync_copy(xt_hbm.at[:, pl.ds(b0, BSTRIPE)], idx_v)
    gsems = (sem, sem2)
    wsems = (semw, semw2)

    def gather_unit(s, half, gbuf):
        return [
            pltpu.async_copy(
                emb_hbm.at[idx_v.at[
                    s, pl.ds(half * HSTRIPE + h * GCHUNK, GCHUNK)]],
                g_v.at[gbuf, pl.ds(h * GCHUNK, GCHUNK)],
                gsems[gbuf],
            )
            for h in range(HSTRIPE // GCHUNK)
        ]

    def transpose_unit(buf, s, half, first):
        boff = b0 + half * HSTRIPE
        dst = out_hbm.at[s, :, pl.ds(boff, HSTRIPE)]
        osrc = o_v.at[buf, :, pl.ds(0, HSTRIPE)]

        @pl.when(jnp.logical_not(first))
        def _():
            pltpu.make_async_copy(osrc, dst, wsems[buf]).wait()

        ob = o_v.at[buf]

        def row_body(j, c2):
            for cg in range(D // LANES):
                vals = g_v[buf, j, pl.ds(cg * LANES, LANES)] * SCALE
                cvec = lax.iota(jnp.int32, LANES) + cg * LANES
                jvec = jnp.full((LANES,), j, jnp.int32)
                plsc.store_scatter(ob, [cvec, jvec], vals)
            return c2

        lax.fori_loop(0, HSTRIPE, row_body, 0, unroll=8)
        pltpu.async_copy(osrc, dst, wsems[buf])

    for cp in gather_unit(0, 0, 0):
        cp.wait()

    def t_body(tt, c3):
        cps = gather_unit(tt, 1, 1)
        transpose_unit(0, tt, 0, tt == 0)
        for cp in cps:
            cp.wait()

        @pl.when(tt + 1 < S)
        def _():
            cps2 = gather_unit(tt + 1, 0, 0)
            transpose_unit(1, tt, 1, tt == 0)
            for cp in cps2:
                cp.wait()

        @pl.when(tt + 1 == S)
        def _():
            transpose_unit(1, tt, 1, tt == 0)

        return c3

    lax.fori_loop(0, S, t_body, 0)
    pltpu.make_async_copy(
        o_v.at[0, :, pl.ds(0, HSTRIPE)],
        out_hbm.at[S - 1, :, pl.ds(b0, HSTRIPE)],
        semw,
    ).wait()
    pltpu.make_async_copy(
        o_v.at[1, :, pl.ds(0, HSTRIPE)],
        out_hbm.at[S - 1, :, pl.ds(b0 + HSTRIPE, HSTRIPE)],
        semw2,
    ).wait()


@jax.jit
def _lookup(xt, emb):
    mesh = plsc.VectorSubcoreMesh(core_axis_name="c", subcore_axis_name="s")
    f = pl.kernel(
        _body,
        out_type=jax.ShapeDtypeStruct((S, D, B), jnp.float32),
        mesh=mesh,
        scratch_types=[
            pltpu.VMEM((S, BSTRIPE), jnp.int32),
            pltpu.VMEM((2, HSTRIPE, D), jnp.float32),
            pltpu.VMEM((2, D, HSTRIPE + 1), jnp.float32),
            pltpu.SemaphoreType.DMA,
            pltpu.SemaphoreType.DMA,
            pltpu.SemaphoreType.DMA,
            pltpu.SemaphoreType.DMA,
        ],
        compiler_params=pltpu.CompilerParams(
            use_tc_tiling_on_sc=False, needs_layout_passes=False),
    )
    return f(xt, emb)


def kernel(x, emb):
    xt = x.T.astype(jnp.int32)
    out3 = _lookup(xt, emb)
    return out3.transpose(2, 0, 1)
